# baseline (speedup 1.0000x reference)
import functools

import jax
import jax.numpy as jnp
from jax import lax
from jax.experimental import pallas as pl
from jax.experimental.pallas import tpu as pltpu
from jax.experimental.pallas import tpu_sc as plsc

VOCAB = 1_000_000
D = 64
DP = 128
EOI = 256000
SCALE = float(D) ** 0.5

NC, NS, LANES = 2, 16, 16
NW = NC * NS
CHUNK = 512
TBLK = 512


def _tc_prep(embT):
    def body(x_ref, eye_ref, o_ref):
        xt = lax.dot_general(
            x_ref[...], eye_ref[...], (((0,), (0,)), ((), ())),
            preferred_element_type=jnp.float32,
        )
        o_ref[:, pl.ds(0, D)] = xt
        o_ref[:, pl.ds(D, D)] = xt

    grid = (VOCAB + TBLK - 1) // TBLK
    return pl.pallas_call(
        body,
        out_shape=jax.ShapeDtypeStruct((VOCAB, DP), jnp.float32),
        grid=(grid,),
        in_specs=[
            pl.BlockSpec((D, TBLK), lambda i: (0, i)),
            pl.BlockSpec((D, D), lambda i: (0, 0)),
        ],
        out_specs=pl.BlockSpec((TBLK, DP), lambda i: (i, 0)),
    )(embT, jnp.eye(D, dtype=jnp.float32) * SCALE)


def _splat(vec, idx):
    return lax.gather(
        vec,
        idx,
        lax.GatherDimensionNumbers(
            offset_dims=(), collapsed_slice_dims=(0,), start_index_map=(0,)
        ),
        (1,),
        mode=lax.GatherScatterMode.PROMISE_IN_BOUNDS,
    )


def _sc_embed(n_ids, table, ids, eoi):
    bpw = n_ids // NW
    nchunk = bpw // CHUNK
    mesh = plsc.VectorSubcoreMesh(core_axis_name="c", subcore_axis_name="s")

    @functools.partial(
        pl.kernel,
        out_type=jax.ShapeDtypeStruct((n_ids, D), jnp.float32),
        mesh=mesh,
        compiler_params=pltpu.CompilerParams(use_tc_tiling_on_sc=False),
        scratch_types=[
            pltpu.VMEM((bpw,), jnp.int32),
            pltpu.VMEM((CHUNK, DP), jnp.float32),
            pltpu.VMEM((D,), jnp.float32),
            pltpu.SemaphoreType.DMA,
        ],
    )
    def body(table_hbm, ids_hbm, eoi_hbm, out_hbm, idx_v, rows, eoi_v, sem):
        wid = lax.axis_index("s") * NC + lax.axis_index("c")
        base = wid * bpw
        pltpu.sync_copy(ids_hbm.at[pl.ds(base, bpw)], idx_v)
        pltpu.sync_copy(eoi_hbm, eoi_v)
        eoi_regs = [eoi_v[pl.ds(j * LANES, LANES)] for j in range(D // LANES)]

        def chunk_body(c, carry):
            cbase = c * CHUNK
            pltpu.async_copy(
                table_hbm.at[idx_v.at[pl.ds(cbase, CHUNK)]], rows, sem
            ).wait()

            def group(g, gcarry):
                iv = idx_v[pl.ds(cbase + g * LANES, LANES)]
                bvec = jnp.where(iv == EOI, 1.0, 0.0).astype(jnp.float32)
                avec = 1.0 - bvec
                for r in range(LANES):
                    row = g * LANES + r
                    rsel = jnp.full((LANES, 1), r, jnp.int32)
                    a = _splat(avec, rsel)
                    b = _splat(bvec, rsel)
                    for j in range(D // LANES):
                        sl = pl.ds(j * LANES, LANES)
                        rows[row, sl] = rows[row, sl] * a + eoi_regs[j] * b

                return gcarry

            lax.fori_loop(0, CHUNK // LANES, group, 0)
            pltpu.sync_copy(
                rows.at[:, pl.ds(0, D)],
                out_hbm.at[pl.ds(base + cbase, CHUNK)],
            )
            return carry

        lax.fori_loop(0, nchunk, chunk_body, 0)

    return body(table, ids, eoi)


def kernel(input_ids, embedding, eoi_embedding):
    tpad = _tc_prep(embedding.T)
    ids = input_ids.reshape(-1)
    out = _sc_embed(ids.shape[0], tpad, ids, eoi_embedding)
    return out.reshape(input_ids.shape + (D,))


# speedup vs baseline: 1.7267x; 1.7267x over previous
import functools

import jax
import jax.numpy as jnp
from jax import lax
from jax.experimental import pallas as pl
from jax.experimental.pallas import tpu as pltpu
from jax.experimental.pallas import tpu_sc as plsc

VOCAB = 1_000_000
D = 64
DP = 128
EOI = 256000
SCALE = float(D) ** 0.5

NC, NS, LANES = 2, 16, 16
NW = NC * NS
CHUNK = 512
TBLK = 4096


def _tc_prep(embT):
    def body(x_ref, eye_ref, o_ref):
        xt = lax.dot_general(
            x_ref[...], eye_ref[...], (((0,), (0,)), ((), ())),
            preferred_element_type=jnp.float32,
        )
        o_ref[:, pl.ds(0, D)] = xt
        o_ref[:, pl.ds(D, D)] = xt

    grid = (VOCAB + TBLK - 1) // TBLK
    return pl.pallas_call(
        body,
        out_shape=jax.ShapeDtypeStruct((VOCAB, DP), jnp.float32),
        grid=(grid,),
        in_specs=[
            pl.BlockSpec((D, TBLK), lambda i: (0, i)),
            pl.BlockSpec((D, D), lambda i: (0, 0)),
        ],
        out_specs=pl.BlockSpec((TBLK, DP), lambda i: (i, 0)),
    )(embT, jnp.eye(D, dtype=jnp.float32) * SCALE)


def _splat(vec, idx):
    return lax.gather(
        vec,
        idx,
        lax.GatherDimensionNumbers(
            offset_dims=(), collapsed_slice_dims=(0,), start_index_map=(0,)
        ),
        (1,),
        mode=lax.GatherScatterMode.PROMISE_IN_BOUNDS,
    )


def _sc_embed(n_ids, table, ids, eoi):
    bpw = n_ids // NW
    nchunk = bpw // CHUNK
    mesh = plsc.VectorSubcoreMesh(core_axis_name="c", subcore_axis_name="s")

    @functools.partial(
        pl.kernel,
        out_type=jax.ShapeDtypeStruct((n_ids, D), jnp.float32),
        mesh=mesh,
        compiler_params=pltpu.CompilerParams(use_tc_tiling_on_sc=False),
        scratch_types=[
            pltpu.VMEM((bpw,), jnp.int32),
            pltpu.VMEM((CHUNK, DP), jnp.float32),
            pltpu.VMEM((D,), jnp.float32),
            pltpu.SemaphoreType.DMA,
        ],
    )
    def body(table_hbm, ids_hbm, eoi_hbm, out_hbm, idx_v, rows, eoi_v, sem):
        wid = lax.axis_index("s") * NC + lax.axis_index("c")
        base = wid * bpw
        pltpu.sync_copy(ids_hbm.at[pl.ds(base, bpw)], idx_v)
        pltpu.sync_copy(eoi_hbm, eoi_v)
        eoi_regs = [eoi_v[pl.ds(j * LANES, LANES)] for j in range(D // LANES)]

        def chunk_body(c, carry):
            cbase = c * CHUNK
            pltpu.async_copy(
                table_hbm.at[idx_v.at[pl.ds(cbase, CHUNK)]], rows, sem
            ).wait()

            def group(g, gcarry):
                iv = idx_v[pl.ds(cbase + g * LANES, LANES)]
                bvec = jnp.where(iv == EOI, 1.0, 0.0).astype(jnp.float32)
                avec = 1.0 - bvec
                for r in range(LANES):
                    row = g * LANES + r
                    rsel = jnp.full((LANES, 1), r, jnp.int32)
                    a = _splat(avec, rsel)
                    b = _splat(bvec, rsel)
                    for j in range(D // LANES):
                        sl = pl.ds(j * LANES, LANES)
                        rows[row, sl] = rows[row, sl] * a + eoi_regs[j] * b

                return gcarry

            lax.fori_loop(0, CHUNK // LANES, group, 0)
            pltpu.sync_copy(
                rows.at[:, pl.ds(0, D)],
                out_hbm.at[pl.ds(base + cbase, CHUNK)],
            )
            return carry

        lax.fori_loop(0, nchunk, chunk_body, 0)

    return body(table, ids, eoi)


def kernel(input_ids, embedding, eoi_embedding):
    tpad = _tc_prep(embedding.T)
    ids = input_ids.reshape(-1)
    out = _sc_embed(ids.shape[0], tpad, ids, eoi_embedding)
    return out.reshape(input_ids.shape + (D,))


# speedup vs baseline: 1.8276x; 1.0584x over previous
import functools

import jax
import jax.numpy as jnp
from jax import lax
from jax.experimental import pallas as pl
from jax.experimental.pallas import tpu as pltpu
from jax.experimental.pallas import tpu_sc as plsc

VOCAB = 1_000_000
D = 64
DP = 128
EOI = 256000
SCALE = float(D) ** 0.5

NC, NS, LANES = 2, 16, 16
NW = NC * NS
CHUNK = 512
TBLK = 8192


def _tc_prep(embT):
    def body(x_ref, eye_ref, o_ref):
        xt = lax.dot_general(
            x_ref[...], eye_ref[...], (((0,), (0,)), ((), ())),
            preferred_element_type=jnp.float32,
        )
        o_ref[:, pl.ds(0, D)] = xt
        o_ref[:, pl.ds(D, D)] = xt

    grid = (VOCAB + TBLK - 1) // TBLK
    return pl.pallas_call(
        body,
        out_shape=jax.ShapeDtypeStruct((VOCAB, DP), jnp.float32),
        grid=(grid,),
        in_specs=[
            pl.BlockSpec((D, TBLK), lambda i: (0, i)),
            pl.BlockSpec((D, D), lambda i: (0, 0)),
        ],
        out_specs=pl.BlockSpec((TBLK, DP), lambda i: (i, 0)),
    )(embT, jnp.eye(D, dtype=jnp.float32) * SCALE)


def _splat(vec, idx):
    return lax.gather(
        vec,
        idx,
        lax.GatherDimensionNumbers(
            offset_dims=(), collapsed_slice_dims=(0,), start_index_map=(0,)
        ),
        (1,),
        mode=lax.GatherScatterMode.PROMISE_IN_BOUNDS,
    )


def _sc_embed(n_ids, table, ids, eoi):
    bpw = n_ids // NW
    nchunk = bpw // CHUNK
    mesh = plsc.VectorSubcoreMesh(core_axis_name="c", subcore_axis_name="s")

    @functools.partial(
        pl.kernel,
        out_type=jax.ShapeDtypeStruct((n_ids, D), jnp.float32),
        mesh=mesh,
        compiler_params=pltpu.CompilerParams(use_tc_tiling_on_sc=False),
        scratch_types=[
            pltpu.VMEM((bpw,), jnp.int32),
            pltpu.VMEM((CHUNK, DP), jnp.float32),
            pltpu.VMEM((D,), jnp.float32),
            pltpu.SemaphoreType.DMA,
        ],
    )
    def body(table_hbm, ids_hbm, eoi_hbm, out_hbm, idx_v, rows, eoi_v, sem):
        wid = lax.axis_index("s") * NC + lax.axis_index("c")
        base = wid * bpw
        pltpu.sync_copy(ids_hbm.at[pl.ds(base, bpw)], idx_v)
        pltpu.sync_copy(eoi_hbm, eoi_v)
        eoi_regs = [eoi_v[pl.ds(j * LANES, LANES)] for j in range(D // LANES)]

        def chunk_body(c, carry):
            cbase = c * CHUNK
            pltpu.async_copy(
                table_hbm.at[idx_v.at[pl.ds(cbase, CHUNK)]], rows, sem
            ).wait()

            def group(g, gcarry):
                iv = idx_v[pl.ds(cbase + g * LANES, LANES)]
                bvec = jnp.where(iv == EOI, 1.0, 0.0).astype(jnp.float32)
                avec = 1.0 - bvec
                for r in range(LANES):
                    row = g * LANES + r
                    rsel = jnp.full((LANES, 1), r, jnp.int32)
                    a = _splat(avec, rsel)
                    b = _splat(bvec, rsel)
                    for j in range(D // LANES):
                        sl = pl.ds(j * LANES, LANES)
                        rows[row, sl] = rows[row, sl] * a + eoi_regs[j] * b

                return gcarry

            lax.fori_loop(0, CHUNK // LANES, group, 0)
            pltpu.sync_copy(
                rows.at[:, pl.ds(0, D)],
                out_hbm.at[pl.ds(base + cbase, CHUNK)],
            )
            return carry

        lax.fori_loop(0, nchunk, chunk_body, 0)

    return body(table, ids, eoi)


def kernel(input_ids, embedding, eoi_embedding):
    tpad = _tc_prep(embedding.T)
    ids = input_ids.reshape(-1)
    out = _sc_embed(ids.shape[0], tpad, ids, eoi_embedding)
    return out.reshape(input_ids.shape + (D,))


# speedup vs baseline: 2.1592x; 1.1814x over previous
import functools

import jax
import jax.numpy as jnp
from jax import lax
from jax.experimental import pallas as pl
from jax.experimental.pallas import tpu as pltpu
from jax.experimental.pallas import tpu_sc as plsc

VOCAB = 1_000_000
D = 64
DP = 128
EOI = 256000
SCALE = float(D) ** 0.5

NC, NS, LANES = 2, 16, 16
NW = NC * NS
CHUNK = 256
TBLK = 16384


def _tc_prep(embT):
    def body(x_ref, eye_ref, o_ref):
        xt = lax.dot_general(
            x_ref[...], eye_ref[...], (((0,), (0,)), ((), ())),
            preferred_element_type=jnp.float32,
        )
        o_ref[:, pl.ds(0, D)] = xt
        o_ref[:, pl.ds(D, D)] = xt

    grid = (VOCAB + TBLK - 1) // TBLK
    return pl.pallas_call(
        body,
        out_shape=jax.ShapeDtypeStruct((VOCAB, DP), jnp.float32),
        grid=(grid,),
        in_specs=[
            pl.BlockSpec((D, TBLK), lambda i: (0, i)),
            pl.BlockSpec((D, D), lambda i: (0, 0)),
        ],
        out_specs=pl.BlockSpec((TBLK, DP), lambda i: (i, 0)),
    )(embT, jnp.eye(D, dtype=jnp.float32) * SCALE)


def _splat(vec, idx):
    return lax.gather(
        vec,
        idx,
        lax.GatherDimensionNumbers(
            offset_dims=(), collapsed_slice_dims=(0,), start_index_map=(0,)
        ),
        (1,),
        mode=lax.GatherScatterMode.PROMISE_IN_BOUNDS,
    )


def _sc_embed(n_ids, table, ids, eoi):
    bpw = n_ids // NW
    nchunk = bpw // CHUNK
    mesh = plsc.VectorSubcoreMesh(core_axis_name="c", subcore_axis_name="s")

    @functools.partial(
        pl.kernel,
        out_type=jax.ShapeDtypeStruct((n_ids, D), jnp.float32),
        mesh=mesh,
        compiler_params=pltpu.CompilerParams(use_tc_tiling_on_sc=False),
        scratch_types=[
            pltpu.VMEM((bpw,), jnp.int32),
            pltpu.VMEM((CHUNK, DP), jnp.float32),
            pltpu.VMEM((CHUNK, DP), jnp.float32),
            pltpu.VMEM((D,), jnp.float32),
            pltpu.SemaphoreType.DMA,
            pltpu.SemaphoreType.DMA,
        ],
    )
    def body(table_hbm, ids_hbm, eoi_hbm, out_hbm, idx_v, rows0, rows1,
             eoi_v, sem0, sem1):
        wid = lax.axis_index("s") * NC + lax.axis_index("c")
        base = wid * bpw
        pltpu.sync_copy(ids_hbm.at[pl.ds(base, bpw)], idx_v)
        pltpu.sync_copy(eoi_hbm, eoi_v)
        eoi_regs = [eoi_v[pl.ds(j * LANES, LANES)] for j in range(D // LANES)]

        def gather(c, rows, sem):
            return pltpu.make_async_copy(
                table_hbm.at[idx_v.at[pl.ds(c * CHUNK, CHUNK)]], rows, sem
            )

        def process(c, rows):
            cbase = c * CHUNK

            def group(g, gcarry):
                iv = idx_v[pl.ds(cbase + g * LANES, LANES)]
                bvec = jnp.where(iv == EOI, 1.0, 0.0).astype(jnp.float32)
                avec = 1.0 - bvec
                for r in range(LANES):
                    row = g * LANES + r
                    rsel = jnp.full((LANES, 1), r, jnp.int32)
                    a = _splat(avec, rsel)
                    b = _splat(bvec, rsel)
                    for j in range(D // LANES):
                        sl = pl.ds(j * LANES, LANES)
                        rows[row, sl] = rows[row, sl] * a + eoi_regs[j] * b

                return gcarry

            lax.fori_loop(0, CHUNK // LANES, group, 0)
            pltpu.sync_copy(
                rows.at[:, pl.ds(0, D)],
                out_hbm.at[pl.ds(base + cbase, CHUNK)],
            )

        gather(0, rows0, sem0).start()

        def pair(cc, carry):
            c0 = cc * 2
            gather(c0 + 1, rows1, sem1).start()
            gather(c0, rows0, sem0).wait()
            process(c0, rows0)

            @pl.when(cc + 1 < nchunk // 2)
            def _():
                gather(c0 + 2, rows0, sem0).start()

            gather(c0 + 1, rows1, sem1).wait()
            process(c0 + 1, rows1)
            return carry

        lax.fori_loop(0, nchunk // 2, pair, 0)

    return body(table, ids, eoi)


def kernel(input_ids, embedding, eoi_embedding):
    tpad = _tc_prep(embedding.T)
    ids = input_ids.reshape(-1)
    out = _sc_embed(ids.shape[0], tpad, ids, eoi_embedding)
    return out.reshape(input_ids.shape + (D,))
